# Initial kernel scaffold; baseline (speedup 1.0000x reference)
import jax
import jax.numpy as jnp
from jax.experimental import pallas as pl


def kernel(indices, table):
    raise NotImplementedError("write your pallas kernel here")


# speedup vs baseline: 1.1877x; 1.1877x over previous
import functools

import jax
import jax.numpy as jnp
from jax import lax
from jax.experimental import pallas as pl
from jax.experimental.pallas import tpu as pltpu
from jax.experimental.pallas import tpu_sc as plsc

EMBED_DIM = 32
CHUNK = 128


def _make(total, dim):
    info = plsc.get_sparse_core_info()
    nc, ns = info.num_cores, info.num_subcores
    nw = nc * ns
    nrows = total // CHUNK
    rows_per_w = nrows // nw

    mesh = plsc.VectorSubcoreMesh(core_axis_name="c", subcore_axis_name="s")

    @functools.partial(
        pl.kernel,
        mesh=mesh,
        out_type=jax.ShapeDtypeStruct((nrows, CHUNK, dim), jnp.float32),
        scratch_types=[
            pltpu.VMEM((rows_per_w, CHUNK), jnp.int32),
            pltpu.VMEM((CHUNK, dim), jnp.float32),
            pltpu.SemaphoreType.DMA,
        ],
        compiler_params=pltpu.CompilerParams(use_tc_tiling_on_sc=False),
    )
    def k(idx_hbm, table_hbm, out_hbm, idx_v, rows_v, sem):
        wid = lax.axis_index("s") * nc + lax.axis_index("c")
        base = wid * rows_per_w
        pltpu.sync_copy(idx_hbm.at[pl.ds(base, rows_per_w)], idx_v)

        def body(j, carry):
            pltpu.async_copy(table_hbm.at[idx_v.at[j]], rows_v, sem).wait()
            pltpu.sync_copy(rows_v, out_hbm.at[base + j])
            return carry

        lax.fori_loop(0, rows_per_w, body, 0)

    return k


def kernel(indices, table):
    b, h = indices.shape
    total = b * h
    idx = indices.reshape(total // CHUNK, CHUNK).astype(jnp.int32)
    out = _make(total, EMBED_DIM)(idx, table)
    return out.reshape(b, h, EMBED_DIM)


# speedup vs baseline: 1.3117x; 1.1044x over previous
import functools

import jax
import jax.numpy as jnp
from jax import lax
from jax.experimental import pallas as pl
from jax.experimental.pallas import tpu as pltpu
from jax.experimental.pallas import tpu_sc as plsc

EMBED_DIM = 32
CHUNK = 128
K = 10


def _make(total, dim):
    info = plsc.get_sparse_core_info()
    nc, ns = info.num_cores, info.num_subcores
    nw = nc * ns
    nrows = total // CHUNK
    rows_per_w = nrows // nw
    nch = rows_per_w // K
    assert rows_per_w % K == 0 and nch % 2 == 0

    mesh = plsc.VectorSubcoreMesh(core_axis_name="c", subcore_axis_name="s")

    @functools.partial(
        pl.kernel,
        mesh=mesh,
        out_type=jax.ShapeDtypeStruct((nrows, CHUNK, dim), jnp.float32),
        scratch_types=[
            pltpu.VMEM((rows_per_w, CHUNK), jnp.int32),
            pltpu.VMEM((K, CHUNK, dim), jnp.float32),
            pltpu.VMEM((K, CHUNK, dim), jnp.float32),
            pltpu.SemaphoreType.DMA,
            pltpu.SemaphoreType.DMA,
        ],
        compiler_params=pltpu.CompilerParams(use_tc_tiling_on_sc=False),
    )
    def k(idx_hbm, table_hbm, out_hbm, idx_v, buf0, buf1, sem0, sem1):
        wid = lax.axis_index("s") * nc + lax.axis_index("c")
        base = wid * rows_per_w
        pltpu.sync_copy(idx_hbm.at[pl.ds(base, rows_per_w)], idx_v)

        bufs = (buf0, buf1)
        sems = (sem0, sem1)

        def fire(g, b):
            for j in range(K):
                pltpu.async_copy(
                    table_hbm.at[idx_v.at[g * K + j]], bufs[b].at[j], sems[b]
                )

        def drain(b):
            for j in range(K):
                pltpu.make_async_copy(
                    table_hbm.at[pl.ds(0, CHUNK)], bufs[b].at[j], sems[b]
                ).wait()

        fire(0, 0)

        def pair_body(t, carry):
            for b in range(2):
                g = 2 * t + b

                @pl.when(g + 1 < nch)
                def _():
                    fire(g + 1, 1 - b)

                drain(b)
                pltpu.sync_copy(bufs[b], out_hbm.at[pl.ds(base + g * K, K)])
            return carry

        lax.fori_loop(0, nch // 2, pair_body, 0)

    return k


def kernel(indices, table):
    b, h = indices.shape
    total = b * h
    idx = indices.reshape(total // CHUNK, CHUNK).astype(jnp.int32)
    out = _make(total, EMBED_DIM)(idx, table)
    return out.reshape(b, h, EMBED_DIM)


# speedup vs baseline: 1.6428x; 1.2524x over previous
import functools

import jax
import jax.numpy as jnp
from jax import lax
from jax.experimental import pallas as pl
from jax.experimental.pallas import tpu as pltpu
from jax.experimental.pallas import tpu_sc as plsc

EMBED_DIM = 32
CHUNK = 128
K = 10


def _make(n_hist, n_batch, dim):
    info = plsc.get_sparse_core_info()
    nc, ns, nl = info.num_cores, info.num_subcores, info.num_lanes
    nw = nc * ns
    total = n_hist * n_batch
    nrows = total // CHUNK
    rows_per_w = nrows // nw
    nch = rows_per_w // K
    cph = n_batch // CHUNK
    kt = dim // 8
    assert rows_per_w % K == 0 and nch % 2 == 0 and total % CHUNK == 0

    mesh = plsc.VectorSubcoreMesh(core_axis_name="c", subcore_axis_name="s")

    @functools.partial(
        pl.kernel,
        mesh=mesh,
        out_type=jax.ShapeDtypeStruct((n_hist, kt, cph, 8, CHUNK), jnp.float32),
        scratch_types=[
            pltpu.VMEM((rows_per_w, CHUNK), jnp.int32),
            pltpu.VMEM((K * CHUNK, dim), jnp.float32),
            pltpu.VMEM((K * CHUNK, dim), jnp.float32),
            pltpu.VMEM((kt, 8, CHUNK), jnp.float32),
            pltpu.VMEM((kt, 8, CHUNK), jnp.float32),
            pltpu.SemaphoreType.DMA,
            pltpu.SemaphoreType.DMA,
            pltpu.SemaphoreType.DMA,
        ],
        compiler_params=pltpu.CompilerParams(
            use_tc_tiling_on_sc=False, needs_layout_passes=False
        ),
    )
    def k(idx_hbm, table_hbm, out_hbm, idx_v, buf0, buf1, tb0, tb1, sem0,
          sem1, osem):
        wid = lax.axis_index("s") * nc + lax.axis_index("c")
        base = wid * rows_per_w
        pltpu.sync_copy(idx_hbm.at[pl.ds(base, rows_per_w)], idx_v)

        bufs = (buf0, buf1)
        tbufs = (tb0, tb1)
        sems = (sem0, sem1)
        lanes = lax.iota(jnp.int32, nl)

        def fire(g, b):
            for j in range(K):
                pltpu.async_copy(
                    table_hbm.at[idx_v.at[g * K + j]],
                    bufs[b].at[pl.ds(j * CHUNK, CHUNK)],
                    sems[b],
                )

        def drain(b):
            for j in range(K):
                pltpu.make_async_copy(
                    table_hbm.at[pl.ds(0, CHUNK)],
                    bufs[b].at[pl.ds(j * CHUNK, CHUNK)],
                    sems[b],
                ).wait()

        def owait():
            pltpu.make_async_copy(out_hbm.at[0, :, 0], tbufs[0], osem).wait()

        fire(0, 0)

        def pair_body(t, carry):
            for b in range(2):
                g = 2 * t + b

                @pl.when(g + 1 < nch)
                def _():
                    fire(g + 1, 1 - b)

                drain(b)
                for r in range(K):
                    gq = (base + g * K + r)
                    h = gq // cph
                    c = gq % cph
                    tb = tbufs[r % 2]

                    @pl.when(g * K + r >= 2)
                    def _():
                        owait()

                    def tr_body(d, carry2):
                        dcol = jnp.full((nl,), d, jnp.int32)
                        for bg in range(CHUNK // nl):
                            rows = r * CHUNK + bg * nl + lanes
                            v = plsc.load_gather(bufs[b], [rows, dcol])
                            tb[d // 8, d % 8, pl.ds(bg * nl, nl)] = v
                        return carry2

                    lax.fori_loop(0, dim, tr_body, 0)
                    pltpu.async_copy(tb, out_hbm.at[h, :, c], osem)
            return carry

        lax.fori_loop(0, nch // 2, pair_body, 0)
        owait()
        owait()

    return k


def kernel(indices, table):
    b, h = indices.shape
    total = b * h
    idx = indices.T.reshape(total // CHUNK, CHUNK)
    out5 = _make(h, b, EMBED_DIM)(idx, table)
    return out5.transpose(2, 4, 0, 1, 3).reshape(b, h, EMBED_DIM)


# speedup vs baseline: 2.5609x; 1.5589x over previous
import functools

import jax
import jax.numpy as jnp
from jax import lax
from jax.experimental import pallas as pl
from jax.experimental.pallas import tpu as pltpu
from jax.experimental.pallas import tpu_sc as plsc

EMBED_DIM = 32
CHUNK = 128
K = 10


def _make(n_hist, n_batch, dim):
    info = plsc.get_sparse_core_info()
    nc, ns, nl = info.num_cores, info.num_subcores, info.num_lanes
    nw = nc * ns
    total = n_hist * n_batch
    nrows = total // CHUNK
    rows_per_w = nrows // nw
    nch = rows_per_w // K
    cph = n_batch // CHUNK
    kt = dim // 8
    assert rows_per_w % K == 0 and nch % 2 == 0 and total % CHUNK == 0

    mesh = plsc.VectorSubcoreMesh(core_axis_name="c", subcore_axis_name="s")

    @functools.partial(
        pl.kernel,
        mesh=mesh,
        out_type=jax.ShapeDtypeStruct((n_hist, kt, cph, 8, CHUNK), jnp.float32),
        scratch_types=[
            pltpu.VMEM((rows_per_w, CHUNK), jnp.int32),
            pltpu.VMEM((K * CHUNK, dim), jnp.float32),
            pltpu.VMEM((K * CHUNK, dim), jnp.float32),
            pltpu.VMEM((kt, 8, CHUNK), jnp.float32),
            pltpu.VMEM((kt, 8, CHUNK), jnp.float32),
            pltpu.SemaphoreType.DMA,
            pltpu.SemaphoreType.DMA,
            pltpu.SemaphoreType.DMA,
        ],
        compiler_params=pltpu.CompilerParams(
            use_tc_tiling_on_sc=False, needs_layout_passes=False
        ),
    )
    def k(idx_hbm, table_hbm, out_hbm, idx_v, buf0, buf1, tb0, tb1, sem0,
          sem1, osem):
        wid = lax.axis_index("s") * nc + lax.axis_index("c")
        base = wid * rows_per_w
        pltpu.sync_copy(idx_hbm.at[pl.ds(base, rows_per_w)], idx_v)

        bufs = (buf0, buf1)
        tbufs = (tb0, tb1)
        sems = (sem0, sem1)
        lanes = lax.iota(jnp.int32, nl)

        def fire(g, b):
            for j in range(K):
                pltpu.async_copy(
                    table_hbm.at[idx_v.at[g * K + j]],
                    bufs[b].at[pl.ds(j * CHUNK, CHUNK)],
                    sems[b],
                )

        def drain(b):
            for j in range(K):
                pltpu.make_async_copy(
                    table_hbm.at[pl.ds(0, CHUNK)],
                    bufs[b].at[pl.ds(j * CHUNK, CHUNK)],
                    sems[b],
                ).wait()

        def owait():
            pltpu.make_async_copy(out_hbm.at[0, :, 0], tbufs[0], osem).wait()

        fire(0, 0)

        def pair_body(t, carry):
            for b in range(2):
                g = 2 * t + b

                @pl.when(g + 1 < nch)
                def _():
                    fire(g + 1, 1 - b)

                drain(b)
                for r in range(K):
                    gq = (base + g * K + r)
                    h = gq // cph
                    c = gq % cph
                    tb = tbufs[r % 2]

                    @pl.when(g * K + r >= 2)
                    def _():
                        owait()

                    def tr_body(d0, carry2):
                        dv = lax.bitwise_and(d0 + lanes, dim - 1)
                        i0 = lax.shift_right_logical(dv, 3)
                        i1 = lax.bitwise_and(dv, 7)
                        for row0 in range(0, CHUNK, nl):
                            rows = r * CHUNK + row0 + lanes
                            v = plsc.load_gather(bufs[b], [rows, dv])
                            plsc.store_scatter(tb, [i0, i1, row0 + lanes], v)
                        return carry2

                    lax.fori_loop(0, dim, tr_body, 0)
                    pltpu.async_copy(tb, out_hbm.at[h, :, c], osem)
            return carry

        lax.fori_loop(0, nch // 2, pair_body, 0)
        owait()
        owait()

    return k


def kernel(indices, table):
    b, h = indices.shape
    total = b * h
    idx = indices.T.reshape(total // CHUNK, CHUNK)
    out5 = _make(h, b, EMBED_DIM)(idx, table)
    return out5.transpose(2, 4, 0, 1, 3).reshape(b, h, EMBED_DIM)


# speedup vs baseline: 3.7614x; 1.4688x over previous
import functools

import jax
import jax.numpy as jnp
from jax import lax
from jax.experimental import pallas as pl
from jax.experimental.pallas import tpu as pltpu
from jax.experimental.pallas import tpu_sc as plsc

EMBED_DIM = 32
CHUNK = 128
K = 10


def _make(n_hist, n_batch, dim):
    info = plsc.get_sparse_core_info()
    nc, ns, nl = info.num_cores, info.num_subcores, info.num_lanes
    nw = nc * ns
    total = n_hist * n_batch
    nrows = total // CHUNK
    rows_per_w = nrows // nw
    nch = rows_per_w // K
    cph = n_batch // CHUNK
    kt = dim // 8
    assert rows_per_w % K == 0 and nch % 2 == 0 and total % CHUNK == 0

    mesh = plsc.VectorSubcoreMesh(core_axis_name="c", subcore_axis_name="s")

    @functools.partial(
        pl.kernel,
        mesh=mesh,
        out_type=jax.ShapeDtypeStruct((n_hist, kt, cph, 8, CHUNK), jnp.float32),
        scratch_types=[
            pltpu.VMEM((rows_per_w, CHUNK), jnp.int32),
            pltpu.VMEM((K * CHUNK, dim), jnp.float32),
            pltpu.VMEM((K * CHUNK, dim), jnp.float32),
            pltpu.VMEM((kt, 8, CHUNK), jnp.float32),
            pltpu.VMEM((kt, 8, CHUNK), jnp.float32),
            pltpu.SemaphoreType.DMA,
            pltpu.SemaphoreType.DMA,
            pltpu.SemaphoreType.DMA,
        ],
        compiler_params=pltpu.CompilerParams(
            use_tc_tiling_on_sc=False, needs_layout_passes=False
        ),
    )
    def k(idx_hbm, table_hbm, out_hbm, idx_v, buf0, buf1, tb0, tb1, sem0,
          sem1, osem):
        wid = lax.axis_index("s") * nc + lax.axis_index("c")
        base = wid * rows_per_w
        pltpu.sync_copy(idx_hbm.at[pl.ds(base, rows_per_w)], idx_v)

        bufs = (buf0, buf1)
        tbufs = (tb0, tb1)
        sems = (sem0, sem1)
        lanes = lax.iota(jnp.int32, nl)

        def fire(g, b):
            for j in range(K):
                pltpu.async_copy(
                    table_hbm.at[idx_v.at[g * K + j]],
                    bufs[b].at[pl.ds(j * CHUNK, CHUNK)],
                    sems[b],
                )

        def drain(b):
            for j in range(K):
                pltpu.make_async_copy(
                    table_hbm.at[pl.ds(0, CHUNK)],
                    bufs[b].at[pl.ds(j * CHUNK, CHUNK)],
                    sems[b],
                ).wait()

        def owait():
            pltpu.make_async_copy(out_hbm.at[0, :, 0], tbufs[0], osem).wait()

        fire(0, 0)

        def pair_body(t, carry):
            for b in range(2):
                g = 2 * t + b

                @pl.when(g + 1 < nch)
                def _():
                    fire(g + 1, 1 - b)

                drain(b)
                for r in range(K):
                    gq = (base + g * K + r)
                    h = gq // cph
                    c = gq % cph
                    tb = tbufs[r % 2]

                    @pl.when(g * K + r >= 2)
                    def _():
                        owait()

                    def tr_body(d0, carry2):
                        dv = lax.bitwise_and(d0 + lanes, dim - 1)
                        i0 = lax.shift_right_logical(dv, 3)
                        i1 = lax.bitwise_and(dv, 7)
                        for row0 in range(0, CHUNK, nl):
                            rows = r * CHUNK + row0 + lanes
                            v = plsc.load_gather(bufs[b], [rows, dv])
                            plsc.store_scatter(tb, [i0, i1, row0 + lanes], v)
                        return carry2

                    lax.fori_loop(0, dim, tr_body, 0)
                    pltpu.async_copy(tb, out_hbm.at[h, :, c], osem)
            return carry

        lax.fori_loop(0, nch // 2, pair_body, 0)
        owait()
        owait()

    return k


VCHUNK = 512


def _make_relayout(vocab, dim):
    info = plsc.get_sparse_core_info()
    nc, ns, nl = info.num_cores, info.num_subcores, info.num_lanes
    nw = nc * ns
    vmain = (vocab // VCHUNK) * VCHUNK
    tail = vocab - vmain
    vpad = vmain + 128
    nchr = vmain // VCHUNK
    iters = (nchr + nw - 1) // nw
    iters += iters % 2
    qrows = VCHUNK * dim // 128

    mesh = plsc.VectorSubcoreMesh(core_axis_name="c", subcore_axis_name="s")

    @functools.partial(
        pl.kernel,
        mesh=mesh,
        out_type=jax.ShapeDtypeStruct((vpad * dim // 128, 128), jnp.float32),
        scratch_types=[
            pltpu.VMEM((dim, VCHUNK), jnp.float32),
            pltpu.VMEM((dim, VCHUNK), jnp.float32),
            pltpu.VMEM((qrows, 128), jnp.float32),
            pltpu.VMEM((qrows, 128), jnp.float32),
            pltpu.VMEM((tail * dim // 128, 128), jnp.float32),
            pltpu.SemaphoreType.DMA,
            pltpu.SemaphoreType.DMA,
            pltpu.SemaphoreType.DMA,
            pltpu.SemaphoreType.DMA,
        ],
        compiler_params=pltpu.CompilerParams(
            use_tc_tiling_on_sc=True, needs_layout_passes=False
        ),
    )
    def k(tab_t, tail_hbm, scr, vb0, vb1, ob0, ob1, tv, is0, is1,
          os0, os1):
        wid = lax.axis_index("s") * nc + lax.axis_index("c")
        vbufs = (vb0, vb1)
        obufs = (ob0, ob1)
        isems = (is0, is1)
        osems = (os0, os1)
        lanes = lax.iota(jnp.int32, nl)

        @pl.when(wid == 0)
        def _():
            pltpu.sync_copy(tail_hbm, tv)
            pltpu.sync_copy(
                tv, scr.at[pl.ds(vmain * dim // 128, tail * dim // 128)]
            )

        def fetch(it, u):
            c = (it * 2 + u) * nw + wid

            @pl.when(c < nchr)
            def _():
                pltpu.async_copy(
                    tab_t.at[:, pl.ds(c * VCHUNK, VCHUNK)], vbufs[u], isems[u]
                )

        def transpose(u):
            vb, ob = vbufs[u], obufs[u]

            def tr_body(d0, carry):
                dv = lax.bitwise_and(d0 + lanes, dim - 1)
                for v0 in range(0, VCHUNK, nl):
                    vv = v0 + lanes
                    val = plsc.load_gather(vb, [dv, vv])
                    i0 = lax.shift_right_logical(vv, 2)
                    i1 = lax.bitwise_and(vv, 3) * dim + dv
                    plsc.store_scatter(ob, [i0, i1], val)
                return carry

            lax.fori_loop(0, dim, tr_body, 0)

        fetch(0, 0)

        def pair_body(t, carry):
            for u in range(2):
                it2 = t * 2 + u
                c = it2 * nw + wid

                fetch(t + (1 if u else 0), 1 - u)

                @pl.when(
                    jnp.logical_and(it2 >= 2, (it2 - 2) * nw + wid < nchr)
                )
                def _():
                    pltpu.make_async_copy(
                        scr.at[pl.ds(0, qrows)], obufs[u], osems[u]
                    ).wait()

                @pl.when(c < nchr)
                def _():
                    pltpu.make_async_copy(
                        tab_t.at[:, pl.ds(0, VCHUNK)], vbufs[u], isems[u]
                    ).wait()
                    transpose(u)
                    pltpu.async_copy(
                        obufs[u], scr.at[pl.ds(c * qrows, qrows)], osems[u]
                    )
            return carry

        lax.fori_loop(0, iters // 2, pair_body, 0)

        @pl.when((iters - 2) * nw + wid < nchr)
        def _():
            pltpu.make_async_copy(scr.at[pl.ds(0, qrows)], ob0, os0).wait()

        @pl.when((iters - 1) * nw + wid < nchr)
        def _():
            pltpu.make_async_copy(scr.at[pl.ds(0, qrows)], ob1, os1).wait()

    return k


def kernel(indices, table):
    b, h = indices.shape
    total = b * h
    v, dim = table.shape
    idx = indices.T.reshape(total // CHUNK, CHUNK)
    vmain = (v // VCHUNK) * VCHUNK
    tail2 = table[vmain:].reshape(-1, 128)
    scratch = _make_relayout(v, dim)(table.T, tail2)
    out5 = _make(h, b, dim)(idx, scratch.reshape(-1, dim))
    return out5.transpose(2, 4, 0, 1, 3).reshape(b, h, EMBED_DIM)


# speedup vs baseline: 3.7626x; 1.0003x over previous
import functools

import jax
import jax.numpy as jnp
from jax import lax
from jax.experimental import pallas as pl
from jax.experimental.pallas import tpu as pltpu
from jax.experimental.pallas import tpu_sc as plsc

EMBED_DIM = 32
CHUNK = 128
K = 10


def _make(n_hist, n_batch, dim):
    info = plsc.get_sparse_core_info()
    nc, ns, nl = info.num_cores, info.num_subcores, info.num_lanes
    nw = nc * ns
    total = n_hist * n_batch
    nrows = total // CHUNK
    rows_per_w = nrows // nw
    nch = rows_per_w // K
    cph = n_batch // CHUNK
    kt = dim // 8
    assert rows_per_w % K == 0 and nch % 2 == 0 and total % CHUNK == 0

    mesh = plsc.VectorSubcoreMesh(core_axis_name="c", subcore_axis_name="s")

    @functools.partial(
        pl.kernel,
        mesh=mesh,
        out_type=jax.ShapeDtypeStruct((n_hist, kt, cph, 8, CHUNK), jnp.float32),
        scratch_types=[
            pltpu.VMEM((rows_per_w, CHUNK), jnp.int32),
            pltpu.VMEM((K * CHUNK, dim), jnp.float32),
            pltpu.VMEM((K * CHUNK, dim), jnp.float32),
            pltpu.VMEM((kt, 8, CHUNK), jnp.float32),
            pltpu.VMEM((kt, 8, CHUNK), jnp.float32),
            pltpu.SemaphoreType.DMA,
            pltpu.SemaphoreType.DMA,
            pltpu.SemaphoreType.DMA,
        ],
        compiler_params=pltpu.CompilerParams(
            use_tc_tiling_on_sc=False, needs_layout_passes=False
        ),
    )
    def k(idx_hbm, table_hbm, out_hbm, idx_v, buf0, buf1, tb0, tb1, sem0,
          sem1, osem):
        wid = lax.axis_index("s") * nc + lax.axis_index("c")
        base = wid * rows_per_w
        pltpu.sync_copy(idx_hbm.at[pl.ds(base, rows_per_w)], idx_v)

        bufs = (buf0, buf1)
        tbufs = (tb0, tb1)
        sems = (sem0, sem1)
        lanes = lax.iota(jnp.int32, nl)

        def fire(g, b):
            for j in range(K):
                pltpu.async_copy(
                    table_hbm.at[idx_v.at[g * K + j]],
                    bufs[b].at[pl.ds(j * CHUNK, CHUNK)],
                    sems[b],
                )

        def drain(b):
            for j in range(K):
                pltpu.make_async_copy(
                    table_hbm.at[pl.ds(0, CHUNK)],
                    bufs[b].at[pl.ds(j * CHUNK, CHUNK)],
                    sems[b],
                ).wait()

        def owait():
            pltpu.make_async_copy(out_hbm.at[0, :, 0], tbufs[0], osem).wait()

        fire(0, 0)

        def pair_body(t, carry):
            for b in range(2):
                g = 2 * t + b

                @pl.when(g + 1 < nch)
                def _():
                    fire(g + 1, 1 - b)

                drain(b)
                for r in range(K):
                    gq = (base + g * K + r)
                    h = gq // cph
                    c = gq % cph
                    tb = tbufs[r % 2]

                    @pl.when(g * K + r >= 2)
                    def _():
                        owait()

                    def tr_body(d0, carry2):
                        dv = lax.bitwise_and(d0 + lanes, dim - 1)
                        i0 = lax.shift_right_logical(dv, 3)
                        i1 = lax.bitwise_and(dv, 7)
                        for row0 in range(0, CHUNK, nl):
                            rows = r * CHUNK + row0 + lanes
                            v = plsc.load_gather(bufs[b], [rows, dv])
                            plsc.store_scatter(tb, [i0, i1, row0 + lanes], v)
                        return carry2

                    lax.fori_loop(0, dim, tr_body, 0)
                    pltpu.async_copy(tb, out_hbm.at[h, :, c], osem)
            return carry

        lax.fori_loop(0, nch // 2, pair_body, 0)
        owait()
        owait()

    return k


VCHUNK = 512


def _make_relayout(vocab, dim):
    info = plsc.get_sparse_core_info()
    nc, ns, nl = info.num_cores, info.num_subcores, info.num_lanes
    nw = nc * ns
    vmain = (vocab // VCHUNK) * VCHUNK
    tail = vocab - vmain
    vpad = vmain + 128
    nchr = vmain // VCHUNK
    iters = (nchr + nw - 1) // nw
    iters += iters % 2
    qrows = VCHUNK * dim // 128

    mesh = plsc.VectorSubcoreMesh(core_axis_name="c", subcore_axis_name="s")

    @functools.partial(
        pl.kernel,
        mesh=mesh,
        out_type=jax.ShapeDtypeStruct((vpad * dim // 128, 128), jnp.float32),
        scratch_types=[
            pltpu.VMEM((dim, VCHUNK), jnp.float32),
            pltpu.VMEM((dim, VCHUNK), jnp.float32),
            pltpu.VMEM((qrows, 128), jnp.float32),
            pltpu.VMEM((qrows, 128), jnp.float32),
            pltpu.VMEM((tail * dim // 128, 128), jnp.float32),
            pltpu.SemaphoreType.DMA,
            pltpu.SemaphoreType.DMA,
            pltpu.SemaphoreType.DMA,
            pltpu.SemaphoreType.DMA,
        ],
        compiler_params=pltpu.CompilerParams(
            use_tc_tiling_on_sc=True, needs_layout_passes=False
        ),
    )
    def k(tab_t, tail_hbm, scr, vb0, vb1, ob0, ob1, tv, is0, is1,
          os0, os1):
        wid = lax.axis_index("s") * nc + lax.axis_index("c")
        vbufs = (vb0, vb1)
        obufs = (ob0, ob1)
        isems = (is0, is1)
        osems = (os0, os1)
        lanes = lax.iota(jnp.int32, nl)

        @pl.when(wid == 0)
        def _():
            pltpu.sync_copy(tail_hbm, tv)
            pltpu.sync_copy(
                tv, scr.at[pl.ds(vmain * dim // 128, tail * dim // 128)]
            )

        def fetch(it, u):
            c = (it * 2 + u) * nw + wid

            @pl.when(c < nchr)
            def _():
                pltpu.async_copy(
                    tab_t.at[:, pl.ds(c * VCHUNK, VCHUNK)], vbufs[u], isems[u]
                )

        def transpose(u):
            vb, ob = vbufs[u], obufs[u]

            def tr_body(d0, carry):
                dv = lax.bitwise_and(d0 + lanes, dim - 1)
                for v0 in range(0, VCHUNK, nl):
                    vv = v0 + lanes
                    val = plsc.load_gather(vb, [dv, vv])
                    i0 = lax.shift_right_logical(vv, 2)
                    i1 = lax.bitwise_and(vv, 3) * dim + dv
                    plsc.store_scatter(ob, [i0, i1], val)
                return carry

            lax.fori_loop(0, dim, tr_body, 0)

        fetch(0, 0)

        def pair_body(t, carry):
            for u in range(2):
                it2 = t * 2 + u
                c = it2 * nw + wid

                fetch(t + (1 if u else 0), 1 - u)

                @pl.when(
                    jnp.logical_and(it2 >= 2, (it2 - 2) * nw + wid < nchr)
                )
                def _():
                    pltpu.make_async_copy(
                        scr.at[pl.ds(0, qrows)], obufs[u], osems[u]
                    ).wait()

                @pl.when(c < nchr)
                def _():
                    pltpu.make_async_copy(
                        tab_t.at[:, pl.ds(0, VCHUNK)], vbufs[u], isems[u]
                    ).wait()
                    transpose(u)
                    pltpu.async_copy(
                        obufs[u], scr.at[pl.ds(c * qrows, qrows)], osems[u]
                    )
            return carry

        lax.fori_loop(0, iters // 2, pair_body, 0)

        @pl.when((iters - 2) * nw + wid < nchr)
        def _():
            pltpu.make_async_copy(scr.at[pl.ds(0, qrows)], ob0, os0).wait()

        @pl.when((iters - 1) * nw + wid < nchr)
        def _():
            pltpu.make_async_copy(scr.at[pl.ds(0, qrows)], ob1, os1).wait()

    return k


def kernel(indices, table):
    b, h = indices.shape
    total = b * h
    v, dim = table.shape
    idx = indices.T.reshape(total // CHUNK, CHUNK).astype(jnp.int32)
    vmain = (v // VCHUNK) * VCHUNK
    tail2 = table[vmain:].reshape(-1, 128)
    scratch = _make_relayout(v, dim)(table.T, tail2)
    out5 = _make(h, b, dim)(idx, scratch.reshape(-1, dim))
    return out5.transpose(2, 4, 0, 1, 3).reshape(b, h, EMBED_DIM)
